# Initial kernel scaffold; baseline (speedup 1.0000x reference)
import jax
import jax.numpy as jnp
from jax.experimental import pallas as pl


def kernel(x, router_W, router_b, down_W, down_b, up_W):
    raise NotImplementedError("write your pallas kernel here")


# speedup vs baseline: 2.1419x; 2.1419x over previous
import functools
import math

import jax
import jax.numpy as jnp
from jax.experimental import pallas as pl
from jax.experimental.pallas import tpu as pltpu

B = 4
S = 2048
H = 2048
D = 2048
E = 8

_TS = 256
_TH = 128
_TM = 256

_INV_SQRT2 = 1.0 / math.sqrt(2.0)


def _router_body(x_ref, w_ref, b_ref, rw_ref, acc_ref):
    i = pl.program_id(0)

    @pl.when(i == 0)
    def _init():
        acc_ref[...] = jnp.zeros_like(acc_ref)

    acc_ref[...] += jnp.sum(x_ref[...], axis=1)

    @pl.when(i == pl.num_programs(0) - 1)
    def _finish():
        pooled = acc_ref[...] * (1.0 / S)
        logits = jnp.dot(pooled, w_ref[...], preferred_element_type=jnp.float32)
        logits = logits + b_ref[...]
        m = jnp.max(logits, axis=-1, keepdims=True)
        p = jnp.exp(logits - m)
        rw_ref[...] = p / jnp.sum(p, axis=-1, keepdims=True)


def _router(x, router_W, router_b):
    return pl.pallas_call(
        _router_body,
        grid=(S // _TS,),
        in_specs=[
            pl.BlockSpec((B, _TS, H), lambda i: (0, i, 0)),
            pl.BlockSpec((H, E), lambda i: (0, 0)),
            pl.BlockSpec((1, E), lambda i: (0, 0)),
        ],
        out_specs=pl.BlockSpec((B, E), lambda i: (0, 0)),
        out_shape=jax.ShapeDtypeStruct((B, E), jnp.float32),
        scratch_shapes=[pltpu.VMEM((B, H), jnp.float32)],
        compiler_params=pltpu.CompilerParams(
            dimension_semantics=("arbitrary",)),
    )(x, router_W, router_b.reshape(1, E))


def _merge_down_body(rw_ref, dw_ref, db_ref, wd_ref, bd_ref):
    rw = rw_ref[...]
    acc = None
    for e in range(E):
        c = jnp.sum(rw[:, e])
        t = c * dw_ref[e]
        acc = t if acc is None else acc + t
    wd_ref[...] = acc

    @pl.when(pl.program_id(0) == 0)
    def _bias():
        bacc = None
        for e in range(E):
            c = jnp.sum(rw[:, e])
            t = c * db_ref[e:e + 1, :]
            bacc = t if bacc is None else bacc + t
        bd_ref[...] = bacc


def _merge_down(rw, down_W, down_b):
    return pl.pallas_call(
        _merge_down_body,
        grid=(H // _TH,),
        in_specs=[
            pl.BlockSpec((B, E), lambda i: (0, 0)),
            pl.BlockSpec((E, _TH, D), lambda i: (0, i, 0)),
            pl.BlockSpec((E, D), lambda i: (0, 0)),
        ],
        out_specs=[
            pl.BlockSpec((_TH, D), lambda i: (i, 0)),
            pl.BlockSpec((1, D), lambda i: (0, 0)),
        ],
        out_shape=[
            jax.ShapeDtypeStruct((H, D), jnp.float32),
            jax.ShapeDtypeStruct((1, D), jnp.float32),
        ],
        compiler_params=pltpu.CompilerParams(
            dimension_semantics=("arbitrary",)),
    )(rw, down_W, down_b)


def _merge_up_body(rw_ref, uw_ref, wu_ref):
    rw = rw_ref[...]
    acc = None
    for e in range(E):
        c = jnp.sum(rw[:, e])
        t = c * uw_ref[e]
        acc = t if acc is None else acc + t
    wu_ref[...] = acc


def _merge_up(rw, up_W):
    return pl.pallas_call(
        _merge_up_body,
        grid=(D // _TH,),
        in_specs=[
            pl.BlockSpec((B, E), lambda i: (0, 0)),
            pl.BlockSpec((E, _TH, H), lambda i: (0, i, 0)),
        ],
        out_specs=pl.BlockSpec((_TH, H), lambda i: (i, 0)),
        out_shape=jax.ShapeDtypeStruct((D, H), jnp.float32),
        compiler_params=pltpu.CompilerParams(
            dimension_semantics=("arbitrary",)),
    )(rw, up_W)


def _ffn_body(x_ref, wd_ref, bd_ref, wu_ref, out_ref):
    z = jnp.dot(x_ref[...], wd_ref[...], preferred_element_type=jnp.float32)
    z = z + bd_ref[...]
    z = 0.5 * z * (1.0 + jax.lax.erf(z * _INV_SQRT2))
    out_ref[...] = jnp.dot(z, wu_ref[...], preferred_element_type=jnp.float32)


def _ffn(x2d, wd, bd, wu):
    M = x2d.shape[0]
    return pl.pallas_call(
        _ffn_body,
        grid=(M // _TM,),
        in_specs=[
            pl.BlockSpec((_TM, H), lambda i: (i, 0)),
            pl.BlockSpec((H, D), lambda i: (0, 0)),
            pl.BlockSpec((1, D), lambda i: (0, 0)),
            pl.BlockSpec((D, H), lambda i: (0, 0)),
        ],
        out_specs=pl.BlockSpec((_TM, H), lambda i: (i, 0)),
        out_shape=jax.ShapeDtypeStruct((M, H), jnp.float32),
        compiler_params=pltpu.CompilerParams(
            dimension_semantics=("arbitrary",)),
    )(x2d, wd, bd, wu)


def kernel(x, router_W, router_b, down_W, down_b, up_W):
    rw = _router(x, router_W, router_b)
    wd, bd = _merge_down(rw, down_W, down_b)
    wu = _merge_up(rw, up_W)
    x2d = x.reshape(B * S, H)
    out = _ffn(x2d, wd, bd, wu)
    return out.reshape(B, S, H), rw
